# Initial kernel scaffold; baseline (speedup 1.0000x reference)
import jax
import jax.numpy as jnp
from jax.experimental import pallas as pl


def kernel(xyz, points, Ws, bs, gammas, betas):
    raise NotImplementedError("write your pallas kernel here")


# speedup vs baseline: 2.9440x; 2.9440x over previous
import functools
import jax
import jax.numpy as jnp
from jax.experimental import pallas as pl

B = 8
N = 2048
S = 512
RADII = (0.1, 0.2, 0.4)
KS = (16, 32, 128)
MLPS = ((32, 32, 64), (64, 64, 128), (64, 96, 128))
EPS = 1e-5
R_ROWS = 512

_INTERPRET = False


def _pc(*a, **k):
    return pl.pallas_call(*a, interpret=_INTERPRET, **k)


def _mzeros(shape):
    a = jax.lax.broadcasted_iota(jnp.int32, shape, 0)
    b = jax.lax.broadcasted_iota(jnp.int32, shape, 1)
    return jnp.minimum(a + b, 0)


def _fps_body(xyzc_ref, out_ref):
    xyz = xyzc_ref[...]
    iota = jax.lax.broadcasted_iota(jnp.int32, (B, N), 1)
    iota_s = jax.lax.broadcasted_iota(jnp.int32, (B, S), 1)

    def body(i, carry):
        distance, far, cent = carry
        cent = cent + (iota_s == i).astype(jnp.int32) * far
        oh = (iota == far).astype(jnp.float32)
        centroid = jnp.sum(xyz * oh[:, None, :], axis=2, keepdims=True)
        d = jnp.sum((xyz - centroid) ** 2, axis=1)
        distance = jnp.minimum(distance, d)
        m = jnp.max(distance, axis=1, keepdims=True)
        far = jnp.min(jnp.where(distance >= m, iota, N), axis=1, keepdims=True)
        return distance, far, cent

    dist0 = jnp.full((B, N), 1e10, jnp.float32)
    far0 = jnp.zeros((B, 1), jnp.int32)
    cent0 = _mzeros((B, S))
    _, _, cent = jax.lax.fori_loop(0, S, body, (dist0, far0, cent0))
    out_ref[...] = cent


def _run_fps(xyzc):
    return _pc(
        _fps_body,
        out_shape=jax.ShapeDtypeStruct((B, S), jnp.int32),
    )(xyzc)


def _bq_body(xyzc_ref, xyzr_ref, fps_ref, nx_ref, i1_ref, i2_ref, i3_ref):
    xyzc = xyzc_ref[0]
    xyzr = xyzr_ref[0]
    fps = fps_ref[0]
    iota_sn = jax.lax.broadcasted_iota(jnp.int32, (S, N), 1)
    oh = (iota_sn == fps).astype(jnp.float32)
    nx = jnp.dot(oh, xyzr, preferred_element_type=jnp.float32,
                 precision=jax.lax.Precision.HIGHEST)
    nx_ref[0] = nx
    m = jnp.dot(nx, xyzc, preferred_element_type=jnp.float32)
    snx = jnp.sum(nx * nx, axis=1, keepdims=True)
    sx = jnp.sum(xyzc * xyzc, axis=0, keepdims=True)
    dist = (-2.0 * m + snx) + sx

    ut = (jax.lax.broadcasted_iota(jnp.int32, (N, N), 0)
          <= jax.lax.broadcasted_iota(jnp.int32, (N, N), 1)).astype(jnp.float32)
    for r, K, oref in zip(RADII, KS, (i1_ref, i2_ref, i3_ref)):
        mask = (dist <= r * r).astype(jnp.float32)
        rank = jnp.dot(mask, ut, preferred_element_type=jnp.float32,
                   precision=jax.lax.Precision.HIGHEST)
        first = jnp.sum((rank == 0.0).astype(jnp.int32), axis=1, keepdims=True)
        ctot = rank[:, N - 1:N]
        iota_k = jax.lax.broadcasted_iota(jnp.int32, (S, K), 1)

        def body(k, acc, rank=rank, first=first, ctot=ctot, iota_k=iota_k):
            kf = k.astype(jnp.float32)
            cnt = jnp.sum((rank <= kf).astype(jnp.int32), axis=1, keepdims=True)
            val = jnp.where(kf < ctot, cnt, first)
            return acc + (iota_k == k).astype(jnp.int32) * val

        acc0 = _mzeros((S, K))
        oref[0] = jax.lax.fori_loop(0, K, body, acc0)


def _run_bq(xyzc, xyzr, fps2d):
    outs = (
        jax.ShapeDtypeStruct((B, S, 3), jnp.float32),
        jax.ShapeDtypeStruct((B, S, KS[0]), jnp.int32),
        jax.ShapeDtypeStruct((B, S, KS[1]), jnp.int32),
        jax.ShapeDtypeStruct((B, S, KS[2]), jnp.int32),
    )
    return _pc(
        _bq_body,
        grid=(B,),
        in_specs=[
            pl.BlockSpec((1, 3, N), lambda b: (b, 0, 0)),
            pl.BlockSpec((1, N, 3), lambda b: (b, 0, 0)),
            pl.BlockSpec((1, S, 1), lambda b: (b, 0, 0)),
        ],
        out_specs=[
            pl.BlockSpec((1, S, 3), lambda b: (b, 0, 0)),
            pl.BlockSpec((1, S, KS[0]), lambda b: (b, 0, 0)),
            pl.BlockSpec((1, S, KS[1]), lambda b: (b, 0, 0)),
            pl.BlockSpec((1, S, KS[2]), lambda b: (b, 0, 0)),
        ],
        out_shape=outs,
    )(xyzc, xyzr, fps2d)


def _c1_body(t_ref, idx_ref, nxr_ref, w_ref, b_ref, y_ref, st_ref, *, nsb):
    i_b = pl.program_id(0)
    i_j = pl.program_id(1)
    t = t_ref[0]
    idxf = idx_ref[0]
    iota_rn = jax.lax.broadcasted_iota(jnp.int32, (R_ROWS, N), 1)
    oh = (iota_rn == idxf).astype(jnp.float32)
    g8 = jnp.dot(oh, t, preferred_element_type=jnp.float32,
                 precision=jax.lax.Precision.HIGHEST)
    g8 = g8 - nxr_ref[0]
    y = jnp.dot(g8, w_ref[...], preferred_element_type=jnp.float32) + b_ref[...]
    y_ref[...] = y

    @pl.when(jnp.logical_and(i_b == 0, i_j == 0))
    def _():
        st_ref[...] = jnp.zeros_like(st_ref)

    st_ref[0:1, :] += jnp.sum(y, axis=0, keepdims=True)
    st_ref[1:2, :] += jnp.sum(y * y, axis=0, keepdims=True)


def _run_c1(tbl, idxf, nxrep, w1p, b1, K, c1):
    nsb = S // (R_ROWS // K)
    tot = B * S * K
    return _pc(
        functools.partial(_c1_body, nsb=nsb),
        grid=(B, nsb),
        in_specs=[
            pl.BlockSpec((1, N, 8), lambda b, j: (b, 0, 0)),
            pl.BlockSpec((1, R_ROWS, 1), lambda b, j: (b, j, 0)),
            pl.BlockSpec((1, R_ROWS, 8), lambda b, j: (b, j, 0)),
            pl.BlockSpec((8, c1), lambda b, j: (0, 0)),
            pl.BlockSpec((1, c1), lambda b, j: (0, 0)),
        ],
        out_specs=[
            pl.BlockSpec((R_ROWS, c1), lambda b, j, nsb=nsb: (b * nsb + j, 0)),
            pl.BlockSpec((8, c1), lambda b, j: (0, 0)),
        ],
        out_shape=(
            jax.ShapeDtypeStruct((tot, c1), jnp.float32),
            jax.ShapeDtypeStruct((8, c1), jnp.float32),
        ),
    )(tbl, idxf, nxrep, w1p, b1)


def _c2_body(y_ref, a_ref, c_ref, w_ref, b_ref, o_ref, st_ref):
    i = pl.program_id(0)
    h = jnp.maximum(y_ref[...] * a_ref[...] + c_ref[...], 0.0)
    o = jnp.dot(h, w_ref[...], preferred_element_type=jnp.float32) + b_ref[...]
    o_ref[...] = o

    @pl.when(i == 0)
    def _():
        st_ref[...] = jnp.zeros_like(st_ref)

    st_ref[0:1, :] += jnp.sum(o, axis=0, keepdims=True)
    st_ref[1:2, :] += jnp.sum(o * o, axis=0, keepdims=True)


def _run_c2(y, a, c, w, b, cin, cout):
    tot = y.shape[0]
    return _pc(
        _c2_body,
        grid=(tot // R_ROWS,),
        in_specs=[
            pl.BlockSpec((R_ROWS, cin), lambda i: (i, 0)),
            pl.BlockSpec((1, cin), lambda i: (0, 0)),
            pl.BlockSpec((1, cin), lambda i: (0, 0)),
            pl.BlockSpec((cin, cout), lambda i: (0, 0)),
            pl.BlockSpec((1, cout), lambda i: (0, 0)),
        ],
        out_specs=[
            pl.BlockSpec((R_ROWS, cout), lambda i: (i, 0)),
            pl.BlockSpec((8, cout), lambda i: (0, 0)),
        ],
        out_shape=(
            jax.ShapeDtypeStruct((tot, cout), jnp.float32),
            jax.ShapeDtypeStruct((8, cout), jnp.float32),
        ),
    )(y, a, c, w, b)


def _c3_body(y_ref, a_ref, c_ref, w_ref, b_ref, mx_ref, st_ref, *, K, sb):
    i_b = pl.program_id(0)
    i_j = pl.program_id(1)
    h = jnp.maximum(y_ref[...] * a_ref[...] + c_ref[...], 0.0)
    o = jnp.dot(h, w_ref[...], preferred_element_type=jnp.float32) + b_ref[...]

    @pl.when(jnp.logical_and(i_b == 0, i_j == 0))
    def _():
        st_ref[...] = jnp.zeros_like(st_ref)

    st_ref[0:1, :] += jnp.sum(o, axis=0, keepdims=True)
    st_ref[1:2, :] += jnp.sum(o * o, axis=0, keepdims=True)
    cout = o.shape[1]
    mx_ref[0, 0] = jnp.max(o.reshape(sb, K, cout), axis=1)


def _run_c3(y, a, c, w, b, K, cin, cout):
    sb = R_ROWS // K
    nsb = S // sb
    return _pc(
        functools.partial(_c3_body, K=K, sb=sb),
        grid=(B, nsb),
        in_specs=[
            pl.BlockSpec((R_ROWS, cin), lambda b, j, nsb=nsb: (b * nsb + j, 0)),
            pl.BlockSpec((1, cin), lambda b, j: (0, 0)),
            pl.BlockSpec((1, cin), lambda b, j: (0, 0)),
            pl.BlockSpec((cin, cout), lambda b, j: (0, 0)),
            pl.BlockSpec((1, cout), lambda b, j: (0, 0)),
        ],
        out_specs=[
            pl.BlockSpec((1, 1, sb, cout), lambda b, j: (b, j, 0, 0)),
            pl.BlockSpec((8, cout), lambda b, j: (0, 0)),
        ],
        out_shape=(
            jax.ShapeDtypeStruct((B, nsb, sb, cout), jnp.float32),
            jax.ShapeDtypeStruct((8, cout), jnp.float32),
        ),
    )(y, a, c, w, b)


def _d_body(m1_ref, m2_ref, m3_ref, a1_ref, c1_ref, a2_ref, c2_ref,
            a3_ref, c3_ref, o_ref):
    off = 0
    for m_ref, a_ref, c_ref, cw in (
        (m1_ref, a1_ref, c1_ref, MLPS[0][-1]),
        (m2_ref, a2_ref, c2_ref, MLPS[1][-1]),
        (m3_ref, a3_ref, c3_ref, MLPS[2][-1]),
    ):
        h = jnp.maximum(m_ref[0] * a_ref[...] + c_ref[...], 0.0)
        o_ref[0, :, off:off + cw] = h
        off += cw


def _run_d(m1, m2, m3, scs):
    cws = [m[-1] for m in MLPS]
    return _pc(
        _d_body,
        grid=(B,),
        in_specs=[
            pl.BlockSpec((1, S, cws[0]), lambda b: (b, 0, 0)),
            pl.BlockSpec((1, S, cws[1]), lambda b: (b, 0, 0)),
            pl.BlockSpec((1, S, cws[2]), lambda b: (b, 0, 0)),
        ] + [pl.BlockSpec((1, cw), lambda b: (0, 0))
             for cw in (cws[0], cws[0], cws[1], cws[1], cws[2], cws[2])],
        out_specs=pl.BlockSpec((1, S, 320), lambda b: (b, 0, 0)),
        out_shape=jax.ShapeDtypeStruct((B, S, 320), jnp.float32),
    )(m1, m2, m3, *scs)


def _norm_coef(st, gamma, beta, count):
    mean = st[0] / count
    var = st[1] / count - mean * mean
    a = gamma / jnp.sqrt(var + EPS)
    c = beta - mean * a
    return a[None, :], c[None, :]


@jax.jit
def kernel(xyz, points, Ws, bs, gammas, betas):
    xyzc = xyz
    xyzr = xyz.transpose(0, 2, 1)
    ptsr = points.transpose(0, 2, 1)

    fps = _run_fps(xyzc)
    nx, i1, i2, i3 = _run_bq(xyzc, xyzr, fps[:, :, None])

    tbl = jnp.concatenate(
        [ptsr, xyzr, jnp.zeros((B, N, 2), jnp.float32)], axis=-1)
    nx8 = jnp.pad(nx, ((0, 0), (0, 0), (3, 2)))

    maxes = []
    for bi, (K, mlp, idx) in enumerate(zip(KS, MLPS, (i1, i2, i3))):
        idxf = idx.reshape(B, S * K, 1)
        nxrep = jnp.repeat(nx8, K, axis=1)
        w1p = jnp.zeros((8, mlp[0]), jnp.float32).at[:6].set(Ws[bi][0].T)
        count = float(B * S * K)

        y1, st1 = _run_c1(tbl, idxf, nxrep, w1p, bs[bi][0][None, :], K, mlp[0])
        a1, c1 = _norm_coef(st1, gammas[bi][0], betas[bi][0], count)
        y2, st2 = _run_c2(y1, a1, c1, Ws[bi][1].T, bs[bi][1][None, :],
                          mlp[0], mlp[1])
        a2, c2 = _norm_coef(st2, gammas[bi][1], betas[bi][1], count)
        mx, st3 = _run_c3(y2, a2, c2, Ws[bi][2].T, bs[bi][2][None, :],
                          K, mlp[1], mlp[2])
        a3, c3 = _norm_coef(st3, gammas[bi][2], betas[bi][2], count)
        maxes.append((mx.reshape(B, S, mlp[2]), a3, c3))

    scs = [x for (_, a, c) in maxes for x in (a, c)]
    out = _run_d(maxes[0][0], maxes[1][0], maxes[2][0], scs)
    return nx.transpose(0, 2, 1), out.transpose(0, 2, 1)
